# Initial kernel scaffold; baseline (speedup 1.0000x reference)
import functools

import jax
import jax.numpy as jnp
from jax import lax
from jax.experimental import pallas as pl
from jax.experimental.pallas import tpu as pltpu
from jax.experimental.pallas import tpu_sc as plsc

N = 10000
E = 320000
D = 128

NC = 2
NS = 16
NW = NC * NS
CHUNK = 128
NCH = 80
EDGES_PER_TILE = CHUNK * NCH
E_PAD = EDGES_PER_TILE * NW
N_PAD = 10240
ROWS_PER_TILE = N_PAD // NS
DUMMY = N
RB = 1024
GRID = N_PAD // RB

_sc_mesh = plsc.VectorSubcoreMesh(core_axis_name="c", subcore_axis_name="s")


@functools.partial(
    pl.kernel,
    out_type=jax.ShapeDtypeStruct((NC, N_PAD, 16), jnp.float32),
    mesh=_sc_mesh,
    scratch_types=[
        pltpu.VMEM((NCH, CHUNK), jnp.int32),
        pltpu.VMEM((CHUNK, 16), jnp.float32),
        pltpu.VMEM_SHARED((N_PAD + 16, 16), jnp.float32),
        pltpu.SemaphoreType.DMA,
    ],
)
def _sc_degree(dst_hbm, ones_hbm, zeros_hbm, out_hbm, dst_v, ones_v, acc, sem):
    c = lax.axis_index("c")
    s = lax.axis_index("s")
    wid = s * NC + c
    pltpu.sync_copy(zeros_hbm.at[pl.ds(s * ROWS_PER_TILE, ROWS_PER_TILE)],
                    acc.at[pl.ds(s * ROWS_PER_TILE, ROWS_PER_TILE)])
    pltpu.sync_copy(ones_hbm, ones_v)
    pltpu.sync_copy(dst_hbm.at[wid], dst_v)
    plsc.subcore_barrier()

    def body(i, carry):
        pltpu.sync_copy(ones_v, acc.at[dst_v.at[i]], add=True)
        return carry

    lax.fori_loop(0, NCH, body, 0)
    plsc.subcore_barrier()
    pltpu.sync_copy(acc.at[pl.ds(s * ROWS_PER_TILE, ROWS_PER_TILE)],
                    out_hbm.at[c, pl.ds(s * ROWS_PER_TILE, ROWS_PER_TILE)])


@functools.partial(
    pl.kernel,
    out_type=jax.ShapeDtypeStruct((NC, N_PAD, D), jnp.float32),
    mesh=_sc_mesh,
    scratch_types=[
        pltpu.VMEM((NCH, CHUNK), jnp.int32),
        pltpu.VMEM((NCH, CHUNK), jnp.int32),
        pltpu.VMEM((CHUNK, D), jnp.float32),
        pltpu.VMEM_SHARED((N_PAD + 16, D), jnp.float32),
        pltpu.SemaphoreType.DMA,
    ],
)
def _sc_scatter_rows(g_hbm, src_hbm, dst_hbm, zeros_hbm, out_hbm,
                     src_v, dst_v, rows_v, acc, sem):
    c = lax.axis_index("c")
    s = lax.axis_index("s")
    wid = s * NC + c
    pltpu.sync_copy(zeros_hbm.at[pl.ds(s * ROWS_PER_TILE, ROWS_PER_TILE)],
                    acc.at[pl.ds(s * ROWS_PER_TILE, ROWS_PER_TILE)])
    pltpu.sync_copy(src_hbm.at[wid], src_v)
    pltpu.sync_copy(dst_hbm.at[wid], dst_v)
    plsc.subcore_barrier()

    def body(i, carry):
        pltpu.async_copy(g_hbm.at[src_v.at[i]], rows_v, sem).wait()
        pltpu.sync_copy(rows_v, acc.at[dst_v.at[i]], add=True)
        return carry

    lax.fori_loop(0, NCH, body, 0)
    plsc.subcore_barrier()
    pltpu.sync_copy(acc.at[pl.ds(s * ROWS_PER_TILE, ROWS_PER_TILE)],
                    out_hbm.at[c, pl.ds(s * ROWS_PER_TILE, ROWS_PER_TILE)])


def _deg_scale(h0, h1):
    return lax.rsqrt(h0[:, :1] + h1[:, :1] + 1.0)


def _t1_body(x_ref, w_ref, h0_ref, h1_ref, g_ref):
    d = _deg_scale(h0_ref[...], h1_ref[...])
    h = jnp.dot(x_ref[...], w_ref[...], preferred_element_type=jnp.float32)
    g_ref[...] = h * d


def _t2_body(a0_ref, a1_ref, g_ref, h0_ref, h1_ref, b_ref, w_ref, out_ref):
    d = _deg_scale(h0_ref[...], h1_ref[...])
    z = d * (a0_ref[...] + a1_ref[...] + g_ref[...]) + b_ref[...]
    r = jnp.maximum(z, 0.0)
    out_ref[...] = jnp.dot(r, w_ref[...], preferred_element_type=jnp.float32) * d


def _t3_body(a0_ref, a1_ref, g_ref, h0_ref, h1_ref, b_ref,
             h_ref, logp_ref, pred_ref):
    d = _deg_scale(h0_ref[...], h1_ref[...])
    z = d * (a0_ref[...] + a1_ref[...] + g_ref[...]) + b_ref[...]
    h_ref[...] = z
    m = jnp.max(z, axis=1, keepdims=True)
    lse = m + jnp.log(jnp.sum(jnp.exp(z - m), axis=1, keepdims=True))
    logp_ref[...] = z - lse
    idx = lax.broadcasted_iota(jnp.int32, z.shape, 1)
    pred = jnp.min(jnp.where(z == m, idx, jnp.int32(2**30)), axis=1)
    pred_ref[...] = pred[:, None]


def _row_spec(width=D):
    return pl.BlockSpec((RB, width), lambda i: (i, 0))


def _full_spec(shape):
    return pl.BlockSpec(shape, lambda i: (0,) * len(shape))


def kernel(x, edge_index, W1, b1, W2, b2):
    src = edge_index[0]
    dst = edge_index[1]
    pad = E_PAD - E
    src_p = jnp.concatenate([src, jnp.zeros((pad,), jnp.int32)]).reshape(
        NW, NCH, CHUNK)
    dst_p = jnp.concatenate([dst, jnp.full((pad,), DUMMY, jnp.int32)]).reshape(
        NW, NCH, CHUNK)
    x_pad = jnp.pad(x, ((0, N_PAD - N), (0, 0)))
    ones16 = jnp.ones((CHUNK, 16), jnp.float32)
    zeros16 = jnp.zeros((N_PAD, 16), jnp.float32)
    zerosD = jnp.zeros((N_PAD, D), jnp.float32)
    b1r = b1.reshape(1, D)
    b2r = b2.reshape(1, D)

    hist = _sc_degree(dst_p, ones16, zeros16)
    h0, h1 = hist[0], hist[1]

    g1 = pl.pallas_call(
        _t1_body,
        grid=(GRID,),
        in_specs=[_row_spec(), _full_spec((D, D)), _row_spec(16), _row_spec(16)],
        out_specs=_row_spec(),
        out_shape=jax.ShapeDtypeStruct((N_PAD, D), jnp.float32),
    )(x_pad, W1, h0, h1)

    acc1 = _sc_scatter_rows(g1, src_p, dst_p, zerosD)

    g2 = pl.pallas_call(
        _t2_body,
        grid=(GRID,),
        in_specs=[_row_spec(), _row_spec(), _row_spec(), _row_spec(16),
                  _row_spec(16), _full_spec((1, D)), _full_spec((D, D))],
        out_specs=_row_spec(),
        out_shape=jax.ShapeDtypeStruct((N_PAD, D), jnp.float32),
    )(acc1[0], acc1[1], g1, h0, h1, b1r, W2)

    acc2 = _sc_scatter_rows(g2, src_p, dst_p, zerosD)

    h_out, logp, pred = pl.pallas_call(
        _t3_body,
        grid=(GRID,),
        in_specs=[_row_spec(), _row_spec(), _row_spec(), _row_spec(16),
                  _row_spec(16), _full_spec((1, D))],
        out_specs=[_row_spec(), _row_spec(), _row_spec(1)],
        out_shape=[
            jax.ShapeDtypeStruct((N_PAD, D), jnp.float32),
            jax.ShapeDtypeStruct((N_PAD, D), jnp.float32),
            jax.ShapeDtypeStruct((N_PAD, 1), jnp.int32),
        ],
    )(acc2[0], acc2[1], g2, h0, h1, b2r)

    return (h_out[:N], logp[:N], pred[:N, 0])


# speedup vs baseline: 9.1726x; 9.1726x over previous
import functools

import jax
import jax.numpy as jnp
from jax import lax
from jax.experimental import pallas as pl
from jax.experimental.pallas import tpu as pltpu
from jax.experimental.pallas import tpu_sc as plsc

N = 10000
E = 320000
D = 128

NC = 2
NS = 16
NW = NC * NS
CHUNK = 128
NCH = 80
EDGES_PER_TILE = CHUNK * NCH
E_PAD = EDGES_PER_TILE * NW
N_PAD = 10240
ROWS_PER_TILE = N_PAD // NS
DUMMY = N
NV = EDGES_PER_TILE // 16
RB = 1024
GRID = N_PAD // RB

def _sc_degree_body(dst_hbm, out_hbm, dst_v, hist, sem):
    c = lax.axis_index("c")
    s = lax.axis_index("s")
    wid = s * NC + c
    pltpu.sync_copy(dst_hbm.at[wid], dst_v)

    def zero(i, carry):
        hist[pl.ds(i * 16, 16)] = jnp.zeros((16,), jnp.float32)
        return carry

    lax.fori_loop(0, N_PAD // 16, zero, 0)
    ones = jnp.ones((16,), jnp.float32)

    def body(i, carry):
        plsc.addupdate_scatter(hist, [dst_v[i]], ones)
        return carry

    lax.fori_loop(0, NV, body, 0)
    pltpu.sync_copy(hist, out_hbm.at[wid])


def _sc_scatter_rows_body(g_hbm, src_hbm, dst_hbm, zeros_hbm, out_hbm,
                          src_v, dst_v, rows_v, acc, sem):
    c = lax.axis_index("c")
    s = lax.axis_index("s")
    wid = s * NC + c
    pltpu.sync_copy(zeros_hbm.at[pl.ds(s * ROWS_PER_TILE, ROWS_PER_TILE)],
                    acc.at[pl.ds(s * ROWS_PER_TILE, ROWS_PER_TILE)])
    pltpu.sync_copy(src_hbm.at[wid], src_v)
    pltpu.sync_copy(dst_hbm.at[wid], dst_v)
    plsc.subcore_barrier()

    def body(i, carry):
        pltpu.async_copy(g_hbm.at[src_v.at[i]], rows_v, sem).wait()
        pltpu.sync_copy(rows_v, acc.at[dst_v.at[i]], add=True)
        return carry

    lax.fori_loop(0, NCH, body, 0)
    plsc.subcore_barrier()
    pltpu.sync_copy(acc.at[pl.ds(s * ROWS_PER_TILE, ROWS_PER_TILE)],
                    out_hbm.at[c, pl.ds(s * ROWS_PER_TILE, ROWS_PER_TILE)])


@functools.cache
def _sc_kernels():
    mesh = plsc.VectorSubcoreMesh(
        core_axis_name="c", subcore_axis_name="s",
        num_cores=NC, num_subcores=NS)
    sc_degree = pl.kernel(
        _sc_degree_body,
        out_type=jax.ShapeDtypeStruct((NW, N_PAD), jnp.float32),
        mesh=mesh,
        compiler_params=pltpu.CompilerParams(needs_layout_passes=False),
        scratch_types=[
            pltpu.VMEM((NV, 16), jnp.int32),
            pltpu.VMEM((N_PAD,), jnp.float32),
            pltpu.SemaphoreType.DMA,
        ],
    )
    sc_scatter_rows = pl.kernel(
        _sc_scatter_rows_body,
        out_type=jax.ShapeDtypeStruct((NC, N_PAD, D), jnp.float32),
        mesh=mesh,
        scratch_types=[
            pltpu.VMEM((NCH, CHUNK), jnp.int32),
            pltpu.VMEM((NCH, CHUNK), jnp.int32),
            pltpu.VMEM((CHUNK, D), jnp.float32),
            pltpu.VMEM_SHARED((N_PAD, D), jnp.float32),
            pltpu.SemaphoreType.DMA,
        ],
    )
    return sc_degree, sc_scatter_rows


def _deg_scale(ht):
    return lax.rsqrt(jnp.sum(ht, axis=1, keepdims=True) + 1.0)


def _t1_body(x_ref, w_ref, ht_ref, g_ref):
    d = _deg_scale(ht_ref[...])
    h = jnp.dot(x_ref[...], w_ref[...], preferred_element_type=jnp.float32)
    g_ref[...] = h * d


def _t2_body(a0_ref, a1_ref, g_ref, ht_ref, b_ref, w_ref, out_ref):
    d = _deg_scale(ht_ref[...])
    z = d * (a0_ref[...] + a1_ref[...] + g_ref[...]) + b_ref[...]
    r = jnp.maximum(z, 0.0)
    out_ref[...] = jnp.dot(r, w_ref[...], preferred_element_type=jnp.float32) * d


def _t3_body(a0_ref, a1_ref, g_ref, ht_ref, b_ref,
             h_ref, logp_ref, pred_ref):
    d = _deg_scale(ht_ref[...])
    z = d * (a0_ref[...] + a1_ref[...] + g_ref[...]) + b_ref[...]
    h_ref[...] = z
    m = jnp.max(z, axis=1, keepdims=True)
    lse = m + jnp.log(jnp.sum(jnp.exp(z - m), axis=1, keepdims=True))
    logp_ref[...] = z - lse
    idx = lax.broadcasted_iota(jnp.int32, z.shape, 1)
    pred = jnp.min(jnp.where(z == m, idx, jnp.int32(2**30)), axis=1)
    pred_ref[...] = pred[:, None]


def _row_spec(width=D):
    return pl.BlockSpec((RB, width), lambda i: (i, 0))


def _full_spec(shape):
    return pl.BlockSpec(shape, lambda i: (0,) * len(shape))


def kernel(x, edge_index, W1, b1, W2, b2):
    src = edge_index[0]
    dst = edge_index[1]
    pad = E_PAD - E
    src_p = jnp.concatenate([src, jnp.zeros((pad,), jnp.int32)]).reshape(
        NW, NCH, CHUNK)
    dst_p = jnp.concatenate([dst, jnp.full((pad,), DUMMY, jnp.int32)]).reshape(
        NW, NCH, CHUNK)
    x_pad = jnp.pad(x, ((0, N_PAD - N), (0, 0)))
    zerosD = jnp.zeros((N_PAD, D), jnp.float32)
    b1r = b1.reshape(1, D)
    b2r = b2.reshape(1, D)

    sc_degree, sc_scatter_rows = _sc_kernels()
    hist = sc_degree(dst_p.reshape(NW, NV, 16))
    ht = hist.T

    g1 = pl.pallas_call(
        _t1_body,
        grid=(GRID,),
        in_specs=[_row_spec(), _full_spec((D, D)), _row_spec(NW)],
        out_specs=_row_spec(),
        out_shape=jax.ShapeDtypeStruct((N_PAD, D), jnp.float32),
    )(x_pad, W1, ht)

    acc1 = sc_scatter_rows(g1, src_p, dst_p, zerosD)

    g2 = pl.pallas_call(
        _t2_body,
        grid=(GRID,),
        in_specs=[_row_spec(), _row_spec(), _row_spec(), _row_spec(NW),
                  _full_spec((1, D)), _full_spec((D, D))],
        out_specs=_row_spec(),
        out_shape=jax.ShapeDtypeStruct((N_PAD, D), jnp.float32),
    )(acc1[0], acc1[1], g1, ht, b1r, W2)

    acc2 = sc_scatter_rows(g2, src_p, dst_p, zerosD)

    h_out, logp, pred = pl.pallas_call(
        _t3_body,
        grid=(GRID,),
        in_specs=[_row_spec(), _row_spec(), _row_spec(), _row_spec(NW),
                  _full_spec((1, D))],
        out_specs=[_row_spec(), _row_spec(), _row_spec(1)],
        out_shape=[
            jax.ShapeDtypeStruct((N_PAD, D), jnp.float32),
            jax.ShapeDtypeStruct((N_PAD, D), jnp.float32),
            jax.ShapeDtypeStruct((N_PAD, 1), jnp.int32),
        ],
    )(acc2[0], acc2[1], g2, ht, b2r)

    return (h_out[:N], logp[:N], pred[:N, 0])
